# baseline (speedup 1.0000x reference)
import functools

import jax
import jax.numpy as jnp
from jax import lax
from jax.experimental import pallas as pl
from jax.experimental.pallas import tpu as pltpu
from jax.experimental.pallas import tpu_sc as plsc

NPAD = 10240
NREAL = 10000
B = 64
K = 128
CH = 256
CHE = 2048


def _pad_edges(src, dst, epad):
    e = src.shape[0]
    npadd = epad - e
    ar = jnp.arange(npadd, dtype=jnp.int32)
    src_p = jnp.concatenate([src, ar % NREAL])
    dst_p = jnp.concatenate([dst, NREAL + ar % (NPAD - NREAL)])
    return src_p, dst_p


def _embed_kernel(ea_ref, wt_ref, o_ref):
    o_ref[0, 0] = jnp.dot(ea_ref[0], wt_ref[0],
                          preferred_element_type=jnp.float32)


def _embed(ea_q, wet, nq, epad, dh, esplit):
    nch = epad // CHE
    nc = 1 if esplit else 2
    grid = (nq, nc, nch)
    if esplit:
        wet2 = wet[None]
    else:
        wet2 = jnp.stack([wet[:, :dh], wet[:, dh:]])
    return pl.pallas_call(
        _embed_kernel,
        grid=grid,
        in_specs=[
            pl.BlockSpec((1, CHE, 20), lambda q, c, i: (q, i, 0)),
            pl.BlockSpec((1, 20, dh), lambda q, c, i: (c, 0, 0)),
        ],
        out_specs=pl.BlockSpec((1, 1, CHE, dh), lambda q, c, i: (q, c, i, 0)),
        out_shape=jax.ShapeDtypeStruct((nq, nc, epad, dh), jnp.float32),
    )(ea_q.reshape(nq, epad, 20), wet2)


def _edge_sc(x_all, e_all, src_all, dst_all, scale_all, shift_all,
             nq, epad, dh, esplit):
    rpt = NPAD // 16
    ept = epad // (32 if esplit else 16)
    nchunks = ept // K
    ZR = 16
    mesh = plsc.VectorSubcoreMesh(core_axis_name="c", subcore_axis_name="s")

    @functools.partial(
        pl.kernel,
        out_type=jax.ShapeDtypeStruct((nq * 2 * NPAD, dh), jnp.float32),
        mesh=mesh,
        scratch_types=[
            pltpu.VMEM((K,), jnp.int32),
            pltpu.VMEM((K,), jnp.int32),
            pltpu.VMEM((K, dh), jnp.float32),
            pltpu.VMEM((K, dh), jnp.float32),
            pltpu.VMEM((dh,), jnp.float32),
            pltpu.VMEM((dh,), jnp.float32),
            pltpu.VMEM((ZR, dh), jnp.float32),
            pltpu.VMEM_SHARED((NPAD, dh), jnp.float32),
            pltpu.SemaphoreType.DMA,
        ],
    )
    def kern(x_hbm, e_hbm, src_hbm, dst_hbm, sc_hbm, sh_hbm, out_hbm,
             src_v, dst_v, x_v, e_v, scale_v, shift_v, z_v, aggr_sh, sem):
        c = lax.axis_index("c")
        s = lax.axis_index("s")
        row0 = s * rpt

        def zrow(i, carry):
            for j in range(dh // 16):
                z_v[i, pl.ds(j * 16, 16)] = jnp.zeros((16,), jnp.float32)
            return carry
        lax.fori_loop(0, ZR, zrow, 0)

        def qbody(q, carry):
            if esplit:
                xoff = q * NPAD
                eoff = q * epad
                soff = q * epad + (c * 16 + s) * ept
                pltpu.sync_copy(sc_hbm.at[pl.ds(q * dh, dh)], scale_v)
                pltpu.sync_copy(sh_hbm.at[pl.ds(q * dh, dh)], shift_v)
            else:
                xoff = (2 * q + c) * NPAD
                eoff = (2 * q + c) * epad
                soff = q * epad + s * ept
                pltpu.sync_copy(sc_hbm.at[pl.ds((2 * q + c) * dh, dh)],
                                scale_v)
                pltpu.sync_copy(sh_hbm.at[pl.ds((2 * q + c) * dh, dh)],
                                shift_v)
            for r in range(rpt // ZR):
                pltpu.sync_copy(z_v, aggr_sh.at[pl.ds(row0 + r * ZR, ZR)])
            plsc.subcore_barrier()

            def chunk(g, carry2):
                base = soff + g * K
                pltpu.sync_copy(src_hbm.at[pl.ds(base, K)], src_v)
                pltpu.sync_copy(dst_hbm.at[pl.ds(base, K)], dst_v)

                def offb(i, cc):
                    sl = pl.ds(i * 16, 16)
                    src_v[sl] = src_v[sl] + xoff
                    return cc
                lax.fori_loop(0, K // 16, offb, 0, unroll=True)
                pltpu.async_copy(x_hbm.at[src_v], x_v, sem).wait()
                pltpu.sync_copy(
                    e_hbm.at[pl.ds(eoff + soff - q * epad + g * K, K)], e_v)
                for j in range(dh // 16):
                    jsl = pl.ds(j * 16, 16)
                    sv = scale_v[jsl]
                    hv = shift_v[jsl]

                    def ebody(i, cc):
                        x_v[i, jsl] = jnp.maximum(
                            x_v[i, jsl] * sv + hv + e_v[i, jsl], 0.0)
                        return cc
                    lax.fori_loop(0, K, ebody, 0, unroll=8)
                pltpu.sync_copy(x_v, aggr_sh.at[dst_v], add=True)
                return carry2
            lax.fori_loop(0, nchunks, chunk, 0)
            plsc.subcore_barrier()
            pltpu.sync_copy(
                aggr_sh.at[pl.ds(row0, rpt)],
                out_hbm.at[pl.ds((2 * q + c) * NPAD + row0, rpt)])
            plsc.subcore_barrier()
            return carry
        lax.fori_loop(0, nq, qbody, 0)

    return kern(x_all, e_all, src_all, dst_all, scale_all, shift_all)


def _passb_kernel(esplit, xz_ref, ag_ref, sb_ref, hb_ref, w1_ref, b1_ref,
                  w2_ref, b2_ref, bt_ref,
                  z_ref, ssum_ref, cnt_ref):
    i = pl.program_id(1)
    x = jnp.concatenate([xz_ref[0, 0], xz_ref[0, 1]], axis=-1)
    if esplit:
        ag = ag_ref[0, 0] + ag_ref[0, 1]
    else:
        ag = jnp.concatenate([ag_ref[0, 0], ag_ref[0, 1]], axis=-1)
    h = x * sb_ref[0] + hb_ref[0] + ag
    a = jnp.maximum(jnp.dot(h, w1_ref[...],
                            preferred_element_type=jnp.float32) + b1_ref[0], 0.0)
    z = jnp.maximum(jnp.dot(a, w2_ref[...],
                            preferred_element_type=jnp.float32) + b2_ref[0], 0.0)
    z_ref[0, 0] = z[:, :128]
    z_ref[0, 1] = z[:, 128:]
    rows = i * CH + lax.broadcasted_iota(jnp.int32, (CH, 1), 0)
    valid = rows < NREAL
    zm = jnp.where(valid, z, 0.0)
    bt = bt_ref[0, 0, 0]
    oh = jnp.where((bt[:, None] == lax.broadcasted_iota(jnp.int32, (CH, B), 1))
                   & valid, 1.0, 0.0)

    @pl.when(i == 0)
    def _():
        ssum_ref[...] = jnp.zeros_like(ssum_ref)
        cnt_ref[...] = jnp.zeros_like(cnt_ref)

    ssum_ref[0, 0:1, :] += jnp.sum(zm, axis=0, keepdims=True)
    cnt_ref[0, 0:1, :] += jnp.sum(oh, axis=0, keepdims=True)


def _passb(xz, aggr, scale_b, shift_b, w1t, b1, w2t, b2, bt3, nq, d, esplit):
    dh = d // 2
    nch = NPAD // CH
    grid = (nq, nch)
    out_shapes = (
        jax.ShapeDtypeStruct((nq, 2, NPAD, 128), jnp.float32),
        jax.ShapeDtypeStruct((nq, 8, 256), jnp.float32),
        jax.ShapeDtypeStruct((nq, 8, B), jnp.float32),
    )
    return pl.pallas_call(
        functools.partial(_passb_kernel, esplit),
        grid=grid,
        in_specs=[
            pl.BlockSpec((1, 2, CH, dh), lambda q, i: (q, 0, i, 0)),
            pl.BlockSpec((1, 2, CH, 128), lambda q, i: (q, 0, i, 0)),
            pl.BlockSpec((1, 1, d), lambda q, i: (q, 0, 0)),
            pl.BlockSpec((1, 1, d), lambda q, i: (q, 0, 0)),
            pl.BlockSpec((d, 256), lambda q, i: (0, 0)),
            pl.BlockSpec((1, 256), lambda q, i: (0, 0)),
            pl.BlockSpec((256, 256), lambda q, i: (0, 0)),
            pl.BlockSpec((1, 256), lambda q, i: (0, 0)),
            pl.BlockSpec((1, 1, 1, CH), lambda q, i: (q, i, 0, 0)),
        ],
        out_specs=(
            pl.BlockSpec((1, 2, CH, 128), lambda q, i: (q, 0, i, 0)),
            pl.BlockSpec((1, 8, 256), lambda q, i: (q, 0, 0)),
            pl.BlockSpec((1, 8, B), lambda q, i: (q, 0, 0)),
        ),
        out_shape=out_shapes,
    )(xz, aggr, scale_b, shift_b, w1t, b1, w2t, b2, bt3)


def _passc_kernel(z_ref, mu_ref, bt_ref, ssq_ref, sseg_ref):
    i = pl.program_id(1)
    z = jnp.concatenate([z_ref[0, 0], z_ref[0, 1]], axis=-1)
    zc = z - mu_ref[0]
    rows = i * CH + lax.broadcasted_iota(jnp.int32, (CH, 1), 0)
    valid = rows < NREAL
    zcm = jnp.where(valid, zc, 0.0)
    bt = bt_ref[0, 0, 0]
    oh = jnp.where((bt[:, None] == lax.broadcasted_iota(jnp.int32, (CH, B), 1))
                   & valid, 1.0, 0.0)
    seg = lax.dot_general(oh, zcm, (((0,), (0,)), ((), ())),
                          preferred_element_type=jnp.float32,
                          precision=jax.lax.Precision.HIGHEST)

    @pl.when(i == 0)
    def _():
        ssq_ref[...] = jnp.zeros_like(ssq_ref)
        sseg_ref[...] = jnp.zeros_like(sseg_ref)

    ssq_ref[0, 0:1, :] += jnp.sum(zcm * zcm, axis=0, keepdims=True)
    sseg_ref[0] += seg


def _passc(z, mu, bt3, nq):
    nch = NPAD // CH
    grid = (nq, nch)
    return pl.pallas_call(
        _passc_kernel,
        grid=grid,
        in_specs=[
            pl.BlockSpec((1, 2, CH, 128), lambda q, i: (q, 0, i, 0)),
            pl.BlockSpec((1, 1, 256), lambda q, i: (q, 0, 0)),
            pl.BlockSpec((1, 1, 1, CH), lambda q, i: (q, i, 0, 0)),
        ],
        out_specs=(
            pl.BlockSpec((1, 8, 256), lambda q, i: (q, 0, 0)),
            pl.BlockSpec((1, B, 256), lambda q, i: (q, 0, 0)),
        ),
        out_shape=(
            jax.ShapeDtypeStruct((nq, 8, 256), jnp.float32),
            jax.ShapeDtypeStruct((nq, B, 256), jnp.float32),
        ),
    )(z, mu, bt3)


def _head_kernel(pg_ref, pq_ref, gw_ref, gb_ref, qw_ref, qb_ref,
                 wih0_ref, whh0_ref, bih0_ref, bhh0_ref,
                 wih1_ref, whh1_ref, bih1_ref, bhh1_ref,
                 lw_ref, lb_ref, mw_ref, mb_ref, o_ref):
    def leaky(v):
        return jnp.where(v >= 0.0, v, 0.01 * v)

    def dot(a, bm):
        return jnp.dot(a, bm, preferred_element_type=jnp.float32)

    g = leaky(dot(pg_ref[0], gw_ref[...]) + gb_ref[0])
    seq = []
    for t in range(8):
        q = leaky(dot(pq_ref[t], qw_ref[...]) + qb_ref[0])
        seq.append(jnp.concatenate([q, q - g, q * g], axis=1))
    hns = []
    for l in range(2):
        wih = wih0_ref if l == 0 else wih1_ref
        whh = whh0_ref if l == 0 else whh1_ref
        bih = (bih0_ref if l == 0 else bih1_ref)[0]
        bhh = (bhh0_ref if l == 0 else bhh1_ref)[0]
        h = jnp.zeros((B, 256), jnp.float32)
        outs = []
        for t in range(8):
            gi = dot(seq[t], wih[...]) + bih
            gh = dot(h, whh[...]) + bhh
            r = jax.nn.sigmoid(gi[:, :256] + gh[:, :256])
            zz = jax.nn.sigmoid(gi[:, 256:512] + gh[:, 256:512])
            n = jnp.tanh(gi[:, 512:] + r * gh[:, 512:])
            h = (1.0 - zz) * n + zz * h
            outs.append(h)
        seq = outs
        hns.append(h)
    hn_cat = jnp.concatenate(hns, axis=1)
    lo = leaky(dot(hn_cat, lw_ref[...]) + lb_ref[0])
    feat = jnp.concatenate([lo, g], axis=1)
    o_ref[...] = jax.nn.sigmoid(dot(feat, mw_ref[...]) + mb_ref[0])


def _head(pooled_g, pooled_q, params):
    gp, qp, gru, lp, mp = (params["g_proj"], params["q_proj"], params["gru"],
                           params["lstm_proj"], params["matcher"])
    args = (
        pooled_g[None], pooled_q,
        gp["W"].T, gp["b"][None], qp["W"].T, qp["b"][None],
        gru["W_ih"][0].T, gru["W_hh"][0].T, gru["b_ih"][0][None],
        gru["b_hh"][0][None],
        gru["W_ih"][1].T, gru["W_hh"][1].T, gru["b_ih"][1][None],
        gru["b_hh"][1][None],
        lp["W"].T, lp["b"][None], mp["W"].T, mp["b"][None],
    )
    return pl.pallas_call(
        _head_kernel,
        out_shape=jax.ShapeDtypeStruct((B, 1), jnp.float32),
    )(*args)


def _branch(x0, src, dst, ea_pad, bt3, layers, nq, epad):
    dims = [128, 256, 256]
    xz = jnp.stack([x0[:, :, :64], x0[:, :, 64:]], axis=1)
    scale = jnp.ones((nq, 256), jnp.float32)
    shift = jnp.zeros((nq, 256), jnp.float32)
    pooled = []
    for li, p in enumerate(layers):
        d = dims[li]
        dh = d // 2
        esplit = (li == 0)
        epsf = 1.0 + p["eps"]
        if esplit:
            sc_sc = scale[:, :d].reshape(-1)
            sh_sc = (shift[:, :d] + p["be"][None]).reshape(-1)
            x_sc = x0.reshape(nq * NPAD, d)
            e2 = _embed(ea_pad, p["We"].T, nq, epad, d, True)
            e_sc = e2.reshape(nq * epad, d)
        else:
            sc_sc = scale[:, :d].reshape(-1)
            sh_sc = (shift[:, :d]
                     + p["be"][None]).reshape(-1)
            x_sc = xz.reshape(nq * 2 * NPAD, dh)
            e2 = _embed(ea_pad, p["We"].T, nq, epad, dh, False)
            e_sc = e2.reshape(nq * 2 * epad, dh)
        aggr = _edge_sc(x_sc, e_sc, src, dst, sc_sc, sh_sc,
                        nq, epad, d if esplit else dh, esplit)
        sb = (scale[:, :d] * epsf)[:, None, :]
        hb = (shift[:, :d] * epsf)[:, None, :]
        z, ssum, cnt = _passb(
            xz, aggr.reshape(nq, 2, NPAD, 128), sb, hb,
            p["W1"].T, p["b1"][None], p["W2"].T, p["b2"][None], bt3, nq, d,
            esplit)
        mu = ssum[:, 0, :] / NREAL
        ssq, sseg = _passc(z, mu[:, None, :], bt3, nq)
        var = ssq[:, 0, :] / NREAL
        gamma, beta = p["gamma"][None], p["beta"][None]
        scale = gamma / jnp.sqrt(var + 1e-5)
        shift = beta - mu * scale
        pooled.append(sseg * scale[:, None, :]
                      + cnt[:, 0, :, None] * beta[:, None, :])
        xz = z
    return jnp.concatenate(pooled, axis=-1)


def kernel(crg_x, crg_edge_index, crg_edge_x, crg_batch,
           queries_x, queries_edge_index, queries_edge_x, queries_batch,
           params):
    E = crg_edge_index.shape[1]
    EQ = queries_edge_index.shape[2]
    NQ = queries_x.shape[0]
    epad_g = ((E + 4095) // 4096) * 4096
    epad_q = ((EQ + 4095) // 4096) * 4096

    def prep(x, ei, ea, batch, nq, e, epad):
        xp = jnp.zeros((nq, NPAD, 128), jnp.float32)
        xp = xp.at[:, :NREAL].set(x.reshape(nq, NREAL, 128))
        srcs, dsts, eas = [], [], []
        for i in range(nq):
            s_p, d_p = _pad_edges(ei.reshape(nq, 2, e)[i, 0],
                                  ei.reshape(nq, 2, e)[i, 1], epad)
            srcs.append(s_p)
            dsts.append(d_p)
            eas.append(jnp.pad(ea.reshape(nq, e, 20)[i],
                               ((0, epad - e), (0, 0))))
        bt = jnp.pad(batch.reshape(nq, NREAL), ((0, 0), (0, NPAD - NREAL)))
        bt3 = bt.reshape(nq, NPAD // CH, 1, CH)
        return (xp, jnp.concatenate(srcs), jnp.concatenate(dsts),
                jnp.concatenate(eas), bt3)

    xg, src_g, dst_g, ea_g, bt3_g = prep(
        crg_x, crg_edge_index, crg_edge_x, crg_batch, 1, E, epad_g)
    xq, src_q, dst_q, ea_q, bt3_q = prep(
        queries_x, queries_edge_index, queries_edge_x, queries_batch,
        NQ, EQ, epad_q)

    pooled_g = _branch(xg, src_g, dst_g, ea_g, bt3_g,
                       params["g_layers"], 1, epad_g)[0]
    pooled_q = _branch(xq, src_q, dst_q, ea_q, bt3_q,
                       params["q_layers"], NQ, epad_q)

    return _head(pooled_g, pooled_q, params)


# speedup vs baseline: 1.1521x; 1.1521x over previous
import functools

import jax
import jax.numpy as jnp
from jax import lax
from jax.experimental import pallas as pl
from jax.experimental.pallas import tpu as pltpu
from jax.experimental.pallas import tpu_sc as plsc

NPAD = 10240
NREAL = 10000
B = 64
K = 128
CH = 256
CHE = 2048


def _pad_edges(src, dst, epad):
    e = src.shape[0]
    npadd = epad - e
    ar = jnp.arange(npadd, dtype=jnp.int32)
    src_p = jnp.concatenate([src, ar % NREAL])
    dst_p = jnp.concatenate([dst, NREAL + ar % (NPAD - NREAL)])
    return src_p, dst_p


def _embed_kernel(ea_ref, wt_ref, o_ref):
    o_ref[0, 0] = jnp.dot(ea_ref[0], wt_ref[0],
                          preferred_element_type=jnp.float32)


def _embed(ea_q, wet, nq, epad, dh, esplit):
    nch = epad // CHE
    nc = 1 if esplit else 2
    grid = (nq, nc, nch)
    if esplit:
        wet2 = wet[None]
    else:
        wet2 = jnp.stack([wet[:, :dh], wet[:, dh:]])
    return pl.pallas_call(
        _embed_kernel,
        grid=grid,
        in_specs=[
            pl.BlockSpec((1, CHE, 20), lambda q, c, i: (q, i, 0)),
            pl.BlockSpec((1, 20, dh), lambda q, c, i: (c, 0, 0)),
        ],
        out_specs=pl.BlockSpec((1, 1, CHE, dh), lambda q, c, i: (q, c, i, 0)),
        out_shape=jax.ShapeDtypeStruct((nq, nc, epad, dh), jnp.float32),
    )(ea_q.reshape(nq, epad, 20), wet2)


def _edge_sc(x_all, e_all, src_all, dst_all, scale_all, shift_all,
             nq, epad, dh, esplit):
    rpt = NPAD // 16
    ept = epad // (32 if esplit else 16)
    KK = 64
    nchunks = ept // KK
    ngg = nchunks // 2
    ZR = 16
    mesh = plsc.VectorSubcoreMesh(core_axis_name="c", subcore_axis_name="s")

    @functools.partial(
        pl.kernel,
        out_type=jax.ShapeDtypeStruct((nq * 2 * NPAD, dh), jnp.float32),
        mesh=mesh,
        scratch_types=[
            pltpu.VMEM((2, KK), jnp.int32),
            pltpu.VMEM((2, KK), jnp.int32),
            pltpu.VMEM((2, KK, dh), jnp.float32),
            pltpu.VMEM((2, KK, dh), jnp.float32),
            pltpu.VMEM((dh,), jnp.float32),
            pltpu.VMEM((dh,), jnp.float32),
            pltpu.VMEM((ZR, dh), jnp.float32),
            pltpu.VMEM_SHARED((NPAD, dh), jnp.float32),
            pltpu.SemaphoreType.DMA,
            pltpu.SemaphoreType.DMA,
            pltpu.SemaphoreType.DMA,
            pltpu.SemaphoreType.DMA,
            pltpu.SemaphoreType.DMA,
            pltpu.SemaphoreType.DMA,
            pltpu.SemaphoreType.DMA,
            pltpu.SemaphoreType.DMA,
        ],
    )
    def kern(x_hbm, e_hbm, src_hbm, dst_hbm, sc_hbm, sh_hbm, out_hbm,
             src_v, dst_v, x_v, e_v, scale_v, shift_v, z_v, aggr_sh,
             ss0, ss1, sd0, sd1, sx0, sx1, se0, se1):
        c = lax.axis_index("c")
        s = lax.axis_index("s")
        row0 = s * rpt
        ssem = (ss0, ss1)
        dsem = (sd0, sd1)
        xsem = (sx0, sx1)
        esem = (se0, se1)

        def zrow(i, carry):
            for j in range(dh // 16):
                z_v[i, pl.ds(j * 16, 16)] = jnp.zeros((16,), jnp.float32)
            return carry
        lax.fori_loop(0, ZR, zrow, 0)

        def qbody(q, carry):
            if esplit:
                xoff = q * NPAD
                soff = q * epad + (c * 16 + s) * ept
                eloc = soff
                pltpu.sync_copy(sc_hbm.at[pl.ds(q * dh, dh)], scale_v)
                pltpu.sync_copy(sh_hbm.at[pl.ds(q * dh, dh)], shift_v)
            else:
                xoff = (2 * q + c) * NPAD
                soff = q * epad + s * ept
                eloc = (2 * q + c) * epad + s * ept
                pltpu.sync_copy(sc_hbm.at[pl.ds((2 * q + c) * dh, dh)],
                                scale_v)
                pltpu.sync_copy(sh_hbm.at[pl.ds((2 * q + c) * dh, dh)],
                                shift_v)
            for r in range(rpt // ZR):
                pltpu.sync_copy(z_v, aggr_sh.at[pl.ds(row0 + r * ZR, ZR)])
            plsc.subcore_barrier()

            def issue_idx(b, base):
                pltpu.async_copy(src_hbm.at[pl.ds(base, KK)], src_v.at[b],
                                 ssem[b])
                pltpu.async_copy(dst_hbm.at[pl.ds(base, KK)], dst_v.at[b],
                                 dsem[b])

            for b in range(2):
                issue_idx(b, soff + b * KK)

            def chunk2(gg, carry2):
                g0 = gg * 2
                handles = []
                for b in range(2):
                    pltpu.make_async_copy(
                        src_hbm.at[pl.ds(soff, KK)], src_v.at[b],
                        ssem[b]).wait()
                    pltpu.make_async_copy(
                        dst_hbm.at[pl.ds(soff, KK)], dst_v.at[b],
                        dsem[b]).wait()

                    def offb(i, cc, b=b):
                        sl = pl.ds(i * 16, 16)
                        src_v[b, sl] = src_v[b, sl] + xoff
                        return cc
                    lax.fori_loop(0, KK // 16, offb, 0, unroll=True)
                    hx = pltpu.async_copy(x_hbm.at[src_v.at[b]], x_v.at[b],
                                          xsem[b])
                    he = pltpu.async_copy(
                        e_hbm.at[pl.ds(eloc + (g0 + b) * KK, KK)], e_v.at[b],
                        esem[b])
                    handles.append((hx, he))
                for b in range(2):
                    hx, he = handles[b]
                    hx.wait()
                    he.wait()

                    @pl.when(gg < ngg - 1)
                    def _(b=b):
                        issue_idx(b, soff + (g0 + 2 + b) * KK)
                    for j in range(dh // 16):
                        jsl = pl.ds(j * 16, 16)
                        sv = scale_v[jsl]
                        hv = shift_v[jsl]

                        def ebody(i, cc, b=b, jsl=jsl, sv=sv, hv=hv):
                            x_v[b, i, jsl] = jnp.maximum(
                                x_v[b, i, jsl] * sv + hv + e_v[b, i, jsl],
                                0.0)
                            return cc
                        lax.fori_loop(0, KK, ebody, 0, unroll=8)
                    pltpu.sync_copy(x_v.at[b], aggr_sh.at[dst_v.at[b]],
                                    add=True)
                return carry2
            lax.fori_loop(0, ngg, chunk2, 0)
            plsc.subcore_barrier()
            pltpu.sync_copy(
                aggr_sh.at[pl.ds(row0, rpt)],
                out_hbm.at[pl.ds((2 * q + c) * NPAD + row0, rpt)])
            plsc.subcore_barrier()
            return carry
        lax.fori_loop(0, nq, qbody, 0)

    return kern(x_all, e_all, src_all, dst_all, scale_all, shift_all)


def _passb_kernel(esplit, xz_ref, ag_ref, sb_ref, hb_ref, w1_ref, b1_ref,
                  w2_ref, b2_ref, bt_ref,
                  z_ref, ssum_ref, cnt_ref):
    i = pl.program_id(1)
    x = jnp.concatenate([xz_ref[0, 0], xz_ref[0, 1]], axis=-1)
    if esplit:
        ag = ag_ref[0, 0] + ag_ref[0, 1]
    else:
        ag = jnp.concatenate([ag_ref[0, 0], ag_ref[0, 1]], axis=-1)
    h = x * sb_ref[0] + hb_ref[0] + ag
    a = jnp.maximum(jnp.dot(h, w1_ref[...],
                            preferred_element_type=jnp.float32) + b1_ref[0], 0.0)
    z = jnp.maximum(jnp.dot(a, w2_ref[...],
                            preferred_element_type=jnp.float32) + b2_ref[0], 0.0)
    z_ref[0, 0] = z[:, :128]
    z_ref[0, 1] = z[:, 128:]
    rows = i * CH + lax.broadcasted_iota(jnp.int32, (CH, 1), 0)
    valid = rows < NREAL
    zm = jnp.where(valid, z, 0.0)
    bt = bt_ref[0, 0, 0]
    oh = jnp.where((bt[:, None] == lax.broadcasted_iota(jnp.int32, (CH, B), 1))
                   & valid, 1.0, 0.0)

    @pl.when(i == 0)
    def _():
        ssum_ref[...] = jnp.zeros_like(ssum_ref)
        cnt_ref[...] = jnp.zeros_like(cnt_ref)

    ssum_ref[0, 0:1, :] += jnp.sum(zm, axis=0, keepdims=True)
    cnt_ref[0, 0:1, :] += jnp.sum(oh, axis=0, keepdims=True)


def _passb(xz, aggr, scale_b, shift_b, w1t, b1, w2t, b2, bt3, nq, d, esplit):
    dh = d // 2
    nch = NPAD // CH
    grid = (nq, nch)
    out_shapes = (
        jax.ShapeDtypeStruct((nq, 2, NPAD, 128), jnp.float32),
        jax.ShapeDtypeStruct((nq, 8, 256), jnp.float32),
        jax.ShapeDtypeStruct((nq, 8, B), jnp.float32),
    )
    return pl.pallas_call(
        functools.partial(_passb_kernel, esplit),
        grid=grid,
        in_specs=[
            pl.BlockSpec((1, 2, CH, dh), lambda q, i: (q, 0, i, 0)),
            pl.BlockSpec((1, 2, CH, 128), lambda q, i: (q, 0, i, 0)),
            pl.BlockSpec((1, 1, d), lambda q, i: (q, 0, 0)),
            pl.BlockSpec((1, 1, d), lambda q, i: (q, 0, 0)),
            pl.BlockSpec((d, 256), lambda q, i: (0, 0)),
            pl.BlockSpec((1, 256), lambda q, i: (0, 0)),
            pl.BlockSpec((256, 256), lambda q, i: (0, 0)),
            pl.BlockSpec((1, 256), lambda q, i: (0, 0)),
            pl.BlockSpec((1, 1, 1, CH), lambda q, i: (q, i, 0, 0)),
        ],
        out_specs=(
            pl.BlockSpec((1, 2, CH, 128), lambda q, i: (q, 0, i, 0)),
            pl.BlockSpec((1, 8, 256), lambda q, i: (q, 0, 0)),
            pl.BlockSpec((1, 8, B), lambda q, i: (q, 0, 0)),
        ),
        out_shape=out_shapes,
    )(xz, aggr, scale_b, shift_b, w1t, b1, w2t, b2, bt3)


def _passc_kernel(z_ref, mu_ref, bt_ref, ssq_ref, sseg_ref):
    i = pl.program_id(1)
    z = jnp.concatenate([z_ref[0, 0], z_ref[0, 1]], axis=-1)
    zc = z - mu_ref[0]
    rows = i * CH + lax.broadcasted_iota(jnp.int32, (CH, 1), 0)
    valid = rows < NREAL
    zcm = jnp.where(valid, zc, 0.0)
    bt = bt_ref[0, 0, 0]
    oh = jnp.where((bt[:, None] == lax.broadcasted_iota(jnp.int32, (CH, B), 1))
                   & valid, 1.0, 0.0)
    seg = lax.dot_general(oh, zcm, (((0,), (0,)), ((), ())),
                          preferred_element_type=jnp.float32,
                          precision=jax.lax.Precision.HIGHEST)

    @pl.when(i == 0)
    def _():
        ssq_ref[...] = jnp.zeros_like(ssq_ref)
        sseg_ref[...] = jnp.zeros_like(sseg_ref)

    ssq_ref[0, 0:1, :] += jnp.sum(zcm * zcm, axis=0, keepdims=True)
    sseg_ref[0] += seg


def _passc(z, mu, bt3, nq):
    nch = NPAD // CH
    grid = (nq, nch)
    return pl.pallas_call(
        _passc_kernel,
        grid=grid,
        in_specs=[
            pl.BlockSpec((1, 2, CH, 128), lambda q, i: (q, 0, i, 0)),
            pl.BlockSpec((1, 1, 256), lambda q, i: (q, 0, 0)),
            pl.BlockSpec((1, 1, 1, CH), lambda q, i: (q, i, 0, 0)),
        ],
        out_specs=(
            pl.BlockSpec((1, 8, 256), lambda q, i: (q, 0, 0)),
            pl.BlockSpec((1, B, 256), lambda q, i: (q, 0, 0)),
        ),
        out_shape=(
            jax.ShapeDtypeStruct((nq, 8, 256), jnp.float32),
            jax.ShapeDtypeStruct((nq, B, 256), jnp.float32),
        ),
    )(z, mu, bt3)


def _head_kernel(pg_ref, pq_ref, gw_ref, gb_ref, qw_ref, qb_ref,
                 wih0_ref, whh0_ref, bih0_ref, bhh0_ref,
                 wih1_ref, whh1_ref, bih1_ref, bhh1_ref,
                 lw_ref, lb_ref, mw_ref, mb_ref, o_ref):
    def leaky(v):
        return jnp.where(v >= 0.0, v, 0.01 * v)

    def dot(a, bm):
        return jnp.dot(a, bm, preferred_element_type=jnp.float32)

    g = leaky(dot(pg_ref[0], gw_ref[...]) + gb_ref[0])
    seq = []
    for t in range(8):
        q = leaky(dot(pq_ref[t], qw_ref[...]) + qb_ref[0])
        seq.append(jnp.concatenate([q, q - g, q * g], axis=1))
    hns = []
    for l in range(2):
        wih = wih0_ref if l == 0 else wih1_ref
        whh = whh0_ref if l == 0 else whh1_ref
        bih = (bih0_ref if l == 0 else bih1_ref)[0]
        bhh = (bhh0_ref if l == 0 else bhh1_ref)[0]
        h = jnp.zeros((B, 256), jnp.float32)
        outs = []
        for t in range(8):
            gi = dot(seq[t], wih[...]) + bih
            gh = dot(h, whh[...]) + bhh
            r = jax.nn.sigmoid(gi[:, :256] + gh[:, :256])
            zz = jax.nn.sigmoid(gi[:, 256:512] + gh[:, 256:512])
            n = jnp.tanh(gi[:, 512:] + r * gh[:, 512:])
            h = (1.0 - zz) * n + zz * h
            outs.append(h)
        seq = outs
        hns.append(h)
    hn_cat = jnp.concatenate(hns, axis=1)
    lo = leaky(dot(hn_cat, lw_ref[...]) + lb_ref[0])
    feat = jnp.concatenate([lo, g], axis=1)
    o_ref[...] = jax.nn.sigmoid(dot(feat, mw_ref[...]) + mb_ref[0])


def _head(pooled_g, pooled_q, params):
    gp, qp, gru, lp, mp = (params["g_proj"], params["q_proj"], params["gru"],
                           params["lstm_proj"], params["matcher"])
    args = (
        pooled_g[None], pooled_q,
        gp["W"].T, gp["b"][None], qp["W"].T, qp["b"][None],
        gru["W_ih"][0].T, gru["W_hh"][0].T, gru["b_ih"][0][None],
        gru["b_hh"][0][None],
        gru["W_ih"][1].T, gru["W_hh"][1].T, gru["b_ih"][1][None],
        gru["b_hh"][1][None],
        lp["W"].T, lp["b"][None], mp["W"].T, mp["b"][None],
    )
    return pl.pallas_call(
        _head_kernel,
        out_shape=jax.ShapeDtypeStruct((B, 1), jnp.float32),
    )(*args)


def _branch(x0, src, dst, ea_pad, bt3, layers, nq, epad):
    dims = [128, 256, 256]
    xz = jnp.stack([x0[:, :, :64], x0[:, :, 64:]], axis=1)
    scale = jnp.ones((nq, 256), jnp.float32)
    shift = jnp.zeros((nq, 256), jnp.float32)
    pooled = []
    for li, p in enumerate(layers):
        d = dims[li]
        dh = d // 2
        esplit = (li == 0)
        epsf = 1.0 + p["eps"]
        if esplit:
            sc_sc = scale[:, :d].reshape(-1)
            sh_sc = (shift[:, :d] + p["be"][None]).reshape(-1)
            x_sc = x0.reshape(nq * NPAD, d)
            e2 = _embed(ea_pad, p["We"].T, nq, epad, d, True)
            e_sc = e2.reshape(nq * epad, d)
        else:
            sc_sc = scale[:, :d].reshape(-1)
            sh_sc = (shift[:, :d]
                     + p["be"][None]).reshape(-1)
            x_sc = xz.reshape(nq * 2 * NPAD, dh)
            e2 = _embed(ea_pad, p["We"].T, nq, epad, dh, False)
            e_sc = e2.reshape(nq * 2 * epad, dh)
        aggr = _edge_sc(x_sc, e_sc, src, dst, sc_sc, sh_sc,
                        nq, epad, d if esplit else dh, esplit)
        sb = (scale[:, :d] * epsf)[:, None, :]
        hb = (shift[:, :d] * epsf)[:, None, :]
        z, ssum, cnt = _passb(
            xz, aggr.reshape(nq, 2, NPAD, 128), sb, hb,
            p["W1"].T, p["b1"][None], p["W2"].T, p["b2"][None], bt3, nq, d,
            esplit)
        mu = ssum[:, 0, :] / NREAL
        ssq, sseg = _passc(z, mu[:, None, :], bt3, nq)
        var = ssq[:, 0, :] / NREAL
        gamma, beta = p["gamma"][None], p["beta"][None]
        scale = gamma / jnp.sqrt(var + 1e-5)
        shift = beta - mu * scale
        pooled.append(sseg * scale[:, None, :]
                      + cnt[:, 0, :, None] * beta[:, None, :])
        xz = z
    return jnp.concatenate(pooled, axis=-1)


def kernel(crg_x, crg_edge_index, crg_edge_x, crg_batch,
           queries_x, queries_edge_index, queries_edge_x, queries_batch,
           params):
    E = crg_edge_index.shape[1]
    EQ = queries_edge_index.shape[2]
    NQ = queries_x.shape[0]
    epad_g = ((E + 4095) // 4096) * 4096
    epad_q = ((EQ + 4095) // 4096) * 4096

    def prep(x, ei, ea, batch, nq, e, epad):
        xp = jnp.zeros((nq, NPAD, 128), jnp.float32)
        xp = xp.at[:, :NREAL].set(x.reshape(nq, NREAL, 128))
        srcs, dsts, eas = [], [], []
        for i in range(nq):
            s_p, d_p = _pad_edges(ei.reshape(nq, 2, e)[i, 0],
                                  ei.reshape(nq, 2, e)[i, 1], epad)
            srcs.append(s_p)
            dsts.append(d_p)
            eas.append(jnp.pad(ea.reshape(nq, e, 20)[i],
                               ((0, epad - e), (0, 0))))
        bt = jnp.pad(batch.reshape(nq, NREAL), ((0, 0), (0, NPAD - NREAL)))
        bt3 = bt.reshape(nq, NPAD // CH, 1, CH)
        return (xp, jnp.concatenate(srcs), jnp.concatenate(dsts),
                jnp.concatenate(eas), bt3)

    xg, src_g, dst_g, ea_g, bt3_g = prep(
        crg_x, crg_edge_index, crg_edge_x, crg_batch, 1, E, epad_g)
    xq, src_q, dst_q, ea_q, bt3_q = prep(
        queries_x, queries_edge_index, queries_edge_x, queries_batch,
        NQ, EQ, epad_q)

    pooled_g = _branch(xg, src_g, dst_g, ea_g, bt3_g,
                       params["g_layers"], 1, epad_g)[0]
    pooled_q = _branch(xq, src_q, dst_q, ea_q, bt3_q,
                       params["q_layers"], NQ, epad_q)

    return _head(pooled_g, pooled_q, params)


# speedup vs baseline: 1.1958x; 1.0380x over previous
import functools

import jax
import jax.numpy as jnp
from jax import lax
from jax.experimental import pallas as pl
from jax.experimental.pallas import tpu as pltpu
from jax.experimental.pallas import tpu_sc as plsc

NPAD = 10240
NREAL = 10000
B = 64
K = 128
CH = 256
CHE = 2048


def _pad_edges(src, dst, epad):
    e = src.shape[0]
    npadd = epad - e
    ar = jnp.arange(npadd, dtype=jnp.int32)
    src_p = jnp.concatenate([src, ar % NREAL])
    dst_p = jnp.concatenate([dst, NREAL + ar % (NPAD - NREAL)])
    return src_p, dst_p


def _embed_kernel(ea_ref, wt_ref, o_ref):
    o_ref[0, 0] = jnp.dot(ea_ref[0], wt_ref[0],
                          preferred_element_type=jnp.float32)


def _embed(ea_q, wet, nq, epad, dh, esplit):
    nch = epad // CHE
    nc = 1 if esplit else 2
    grid = (nq, nc, nch)
    if esplit:
        wet2 = wet[None]
    else:
        wet2 = jnp.stack([wet[:, :dh], wet[:, dh:]])
    return pl.pallas_call(
        _embed_kernel,
        grid=grid,
        in_specs=[
            pl.BlockSpec((1, CHE, 20), lambda q, c, i: (q, i, 0)),
            pl.BlockSpec((1, 20, dh), lambda q, c, i: (c, 0, 0)),
        ],
        out_specs=pl.BlockSpec((1, 1, CHE, dh), lambda q, c, i: (q, c, i, 0)),
        out_shape=jax.ShapeDtypeStruct((nq, nc, epad, dh), jnp.float32),
    )(ea_q.reshape(nq, epad, 20), wet2)


def _edge_sc(x_all, e_all, src_all, dst_all, scale_all, shift_all,
             nq, epad, dh, esplit):
    rpt = NPAD // 16
    ept = epad // (32 if esplit else 16)
    KK = 64
    nchunks = ept // KK
    ngg = nchunks // 2
    ZR = 16
    mesh = plsc.VectorSubcoreMesh(core_axis_name="c", subcore_axis_name="s")

    @functools.partial(
        pl.kernel,
        out_type=jax.ShapeDtypeStruct((nq * 2 * NPAD, dh), jnp.float32),
        mesh=mesh,
        scratch_types=[
            pltpu.VMEM((2, KK), jnp.int32),
            pltpu.VMEM((2, KK), jnp.int32),
            pltpu.VMEM((2, KK, dh), jnp.float32),
            pltpu.VMEM((2, KK, dh), jnp.float32),
            pltpu.VMEM((dh,), jnp.float32),
            pltpu.VMEM((dh,), jnp.float32),
            pltpu.VMEM((ZR, dh), jnp.float32),
            pltpu.VMEM_SHARED((NPAD, dh), jnp.float32),
            pltpu.SemaphoreType.DMA,
            pltpu.SemaphoreType.DMA,
            pltpu.SemaphoreType.DMA,
            pltpu.SemaphoreType.DMA,
            pltpu.SemaphoreType.DMA,
            pltpu.SemaphoreType.DMA,
            pltpu.SemaphoreType.DMA,
            pltpu.SemaphoreType.DMA,
        ],
    )
    def kern(x_hbm, e_hbm, src_hbm, dst_hbm, sc_hbm, sh_hbm, out_hbm,
             src_v, dst_v, x_v, e_v, scale_v, shift_v, z_v, aggr_sh,
             ss0, ss1, sd0, sd1, sx0, sx1, se0, se1):
        c = lax.axis_index("c")
        s = lax.axis_index("s")
        row0 = s * rpt
        ssem = (ss0, ss1)
        dsem = (sd0, sd1)
        xsem = (sx0, sx1)
        esem = (se0, se1)

        def zrow(i, carry):
            for j in range(dh // 16):
                z_v[i, pl.ds(j * 16, 16)] = jnp.zeros((16,), jnp.float32)
            return carry
        lax.fori_loop(0, ZR, zrow, 0)

        def qbody(q, carry):
            if esplit:
                xoff = q * NPAD
                soff = q * epad + (c * 16 + s) * ept
                eloc = soff
                pltpu.sync_copy(sc_hbm.at[pl.ds(q * dh, dh)], scale_v)
                pltpu.sync_copy(sh_hbm.at[pl.ds(q * dh, dh)], shift_v)
            else:
                xoff = (2 * q + c) * NPAD
                soff = q * epad + s * ept
                eloc = (2 * q + c) * epad + s * ept
                pltpu.sync_copy(sc_hbm.at[pl.ds((2 * q + c) * dh, dh)],
                                scale_v)
                pltpu.sync_copy(sh_hbm.at[pl.ds((2 * q + c) * dh, dh)],
                                shift_v)
            for r in range(rpt // ZR):
                pltpu.sync_copy(z_v, aggr_sh.at[pl.ds(row0 + r * ZR, ZR)])
            plsc.subcore_barrier()

            def issue_idx(b, base):
                pltpu.async_copy(src_hbm.at[pl.ds(base, KK)], src_v.at[b],
                                 ssem[b])
                pltpu.async_copy(dst_hbm.at[pl.ds(base, KK)], dst_v.at[b],
                                 dsem[b])

            for b in range(2):
                issue_idx(b, soff + b * KK)

            def chunk2(gg, carry2):
                g0 = gg * 2
                handles = []
                for b in range(2):
                    pltpu.make_async_copy(
                        src_hbm.at[pl.ds(soff, KK)], src_v.at[b],
                        ssem[b]).wait()
                    pltpu.make_async_copy(
                        dst_hbm.at[pl.ds(soff, KK)], dst_v.at[b],
                        dsem[b]).wait()

                    def offb(i, cc, b=b):
                        sl = pl.ds(i * 16, 16)
                        src_v[b, sl] = src_v[b, sl] + xoff
                        return cc
                    lax.fori_loop(0, KK // 16, offb, 0, unroll=True)
                    hx = pltpu.async_copy(x_hbm.at[src_v.at[b]], x_v.at[b],
                                          xsem[b])
                    he = pltpu.async_copy(
                        e_hbm.at[pl.ds(eloc + (g0 + b) * KK, KK)], e_v.at[b],
                        esem[b])
                    handles.append((hx, he))
                for b in range(2):
                    hx, he = handles[b]
                    hx.wait()
                    he.wait()
                    for j in range(dh // 16):
                        jsl = pl.ds(j * 16, 16)
                        sv = scale_v[jsl]
                        hv = shift_v[jsl]

                        def ebody(i, cc, b=b, jsl=jsl, sv=sv, hv=hv):
                            x_v[b, i, jsl] = jnp.maximum(
                                x_v[b, i, jsl] * sv + hv + e_v[b, i, jsl],
                                0.0)
                            return cc
                        lax.fori_loop(0, KK, ebody, 0, unroll=8)
                    pltpu.sync_copy(x_v.at[b], aggr_sh.at[dst_v.at[b]],
                                    add=True)

                    @pl.when(gg < ngg - 1)
                    def _(b=b):
                        issue_idx(b, soff + (g0 + 2 + b) * KK)
                return carry2
            lax.fori_loop(0, ngg, chunk2, 0)
            plsc.subcore_barrier()
            pltpu.sync_copy(
                aggr_sh.at[pl.ds(row0, rpt)],
                out_hbm.at[pl.ds((2 * q + c) * NPAD + row0, rpt)])
            plsc.subcore_barrier()
            return carry
        lax.fori_loop(0, nq, qbody, 0)

    return kern(x_all, e_all, src_all, dst_all, scale_all, shift_all)


def _passb_kernel(esplit, xz_ref, ag_ref, sb_ref, hb_ref, w1_ref, b1_ref,
                  w2_ref, b2_ref, bt_ref,
                  z_ref, ssum_ref, cnt_ref):
    i = pl.program_id(1)
    x = jnp.concatenate([xz_ref[0, 0], xz_ref[0, 1]], axis=-1)
    if esplit:
        ag = ag_ref[0, 0] + ag_ref[0, 1]
    else:
        ag = jnp.concatenate([ag_ref[0, 0], ag_ref[0, 1]], axis=-1)
    h = x * sb_ref[0] + hb_ref[0] + ag
    a = jnp.maximum(jnp.dot(h, w1_ref[...],
                            preferred_element_type=jnp.float32) + b1_ref[0], 0.0)
    z = jnp.maximum(jnp.dot(a, w2_ref[...],
                            preferred_element_type=jnp.float32) + b2_ref[0], 0.0)
    z_ref[0, 0] = z[:, :128]
    z_ref[0, 1] = z[:, 128:]
    rows = i * CH + lax.broadcasted_iota(jnp.int32, (CH, 1), 0)
    valid = rows < NREAL
    zm = jnp.where(valid, z, 0.0)
    bt = bt_ref[0, 0, 0]
    oh = jnp.where((bt[:, None] == lax.broadcasted_iota(jnp.int32, (CH, B), 1))
                   & valid, 1.0, 0.0)

    @pl.when(i == 0)
    def _():
        ssum_ref[...] = jnp.zeros_like(ssum_ref)
        cnt_ref[...] = jnp.zeros_like(cnt_ref)

    ssum_ref[0, 0:1, :] += jnp.sum(zm, axis=0, keepdims=True)
    cnt_ref[0, 0:1, :] += jnp.sum(oh, axis=0, keepdims=True)


def _passb(xz, aggr, scale_b, shift_b, w1t, b1, w2t, b2, bt3, nq, d, esplit):
    dh = d // 2
    nch = NPAD // CH
    grid = (nq, nch)
    out_shapes = (
        jax.ShapeDtypeStruct((nq, 2, NPAD, 128), jnp.float32),
        jax.ShapeDtypeStruct((nq, 8, 256), jnp.float32),
        jax.ShapeDtypeStruct((nq, 8, B), jnp.float32),
    )
    return pl.pallas_call(
        functools.partial(_passb_kernel, esplit),
        grid=grid,
        in_specs=[
            pl.BlockSpec((1, 2, CH, dh), lambda q, i: (q, 0, i, 0)),
            pl.BlockSpec((1, 2, CH, 128), lambda q, i: (q, 0, i, 0)),
            pl.BlockSpec((1, 1, d), lambda q, i: (q, 0, 0)),
            pl.BlockSpec((1, 1, d), lambda q, i: (q, 0, 0)),
            pl.BlockSpec((d, 256), lambda q, i: (0, 0)),
            pl.BlockSpec((1, 256), lambda q, i: (0, 0)),
            pl.BlockSpec((256, 256), lambda q, i: (0, 0)),
            pl.BlockSpec((1, 256), lambda q, i: (0, 0)),
            pl.BlockSpec((1, 1, 1, CH), lambda q, i: (q, i, 0, 0)),
        ],
        out_specs=(
            pl.BlockSpec((1, 2, CH, 128), lambda q, i: (q, 0, i, 0)),
            pl.BlockSpec((1, 8, 256), lambda q, i: (q, 0, 0)),
            pl.BlockSpec((1, 8, B), lambda q, i: (q, 0, 0)),
        ),
        out_shape=out_shapes,
    )(xz, aggr, scale_b, shift_b, w1t, b1, w2t, b2, bt3)


def _passc_kernel(z_ref, mu_ref, bt_ref, ssq_ref, sseg_ref):
    i = pl.program_id(1)
    z = jnp.concatenate([z_ref[0, 0], z_ref[0, 1]], axis=-1)
    zc = z - mu_ref[0]
    rows = i * CH + lax.broadcasted_iota(jnp.int32, (CH, 1), 0)
    valid = rows < NREAL
    zcm = jnp.where(valid, zc, 0.0)
    bt = bt_ref[0, 0, 0]
    oh = jnp.where((bt[:, None] == lax.broadcasted_iota(jnp.int32, (CH, B), 1))
                   & valid, 1.0, 0.0)
    seg = lax.dot_general(oh, zcm, (((0,), (0,)), ((), ())),
                          preferred_element_type=jnp.float32,
                          precision=jax.lax.Precision.HIGHEST)

    @pl.when(i == 0)
    def _():
        ssq_ref[...] = jnp.zeros_like(ssq_ref)
        sseg_ref[...] = jnp.zeros_like(sseg_ref)

    ssq_ref[0, 0:1, :] += jnp.sum(zcm * zcm, axis=0, keepdims=True)
    sseg_ref[0] += seg


def _passc(z, mu, bt3, nq):
    nch = NPAD // CH
    grid = (nq, nch)
    return pl.pallas_call(
        _passc_kernel,
        grid=grid,
        in_specs=[
            pl.BlockSpec((1, 2, CH, 128), lambda q, i: (q, 0, i, 0)),
            pl.BlockSpec((1, 1, 256), lambda q, i: (q, 0, 0)),
            pl.BlockSpec((1, 1, 1, CH), lambda q, i: (q, i, 0, 0)),
        ],
        out_specs=(
            pl.BlockSpec((1, 8, 256), lambda q, i: (q, 0, 0)),
            pl.BlockSpec((1, B, 256), lambda q, i: (q, 0, 0)),
        ),
        out_shape=(
            jax.ShapeDtypeStruct((nq, 8, 256), jnp.float32),
            jax.ShapeDtypeStruct((nq, B, 256), jnp.float32),
        ),
    )(z, mu, bt3)


def _head_kernel(pg_ref, pq_ref, gw_ref, gb_ref, qw_ref, qb_ref,
                 wih0_ref, whh0_ref, bih0_ref, bhh0_ref,
                 wih1_ref, whh1_ref, bih1_ref, bhh1_ref,
                 lw_ref, lb_ref, mw_ref, mb_ref, o_ref):
    def leaky(v):
        return jnp.where(v >= 0.0, v, 0.01 * v)

    def dot(a, bm):
        return jnp.dot(a, bm, preferred_element_type=jnp.float32)

    g = leaky(dot(pg_ref[0], gw_ref[...]) + gb_ref[0])
    seq = []
    for t in range(8):
        q = leaky(dot(pq_ref[t], qw_ref[...]) + qb_ref[0])
        seq.append(jnp.concatenate([q, q - g, q * g], axis=1))
    hns = []
    for l in range(2):
        wih = wih0_ref if l == 0 else wih1_ref
        whh = whh0_ref if l == 0 else whh1_ref
        bih = (bih0_ref if l == 0 else bih1_ref)[0]
        bhh = (bhh0_ref if l == 0 else bhh1_ref)[0]
        h = jnp.zeros((B, 256), jnp.float32)
        outs = []
        for t in range(8):
            gi = dot(seq[t], wih[...]) + bih
            gh = dot(h, whh[...]) + bhh
            r = jax.nn.sigmoid(gi[:, :256] + gh[:, :256])
            zz = jax.nn.sigmoid(gi[:, 256:512] + gh[:, 256:512])
            n = jnp.tanh(gi[:, 512:] + r * gh[:, 512:])
            h = (1.0 - zz) * n + zz * h
            outs.append(h)
        seq = outs
        hns.append(h)
    hn_cat = jnp.concatenate(hns, axis=1)
    lo = leaky(dot(hn_cat, lw_ref[...]) + lb_ref[0])
    feat = jnp.concatenate([lo, g], axis=1)
    o_ref[...] = jax.nn.sigmoid(dot(feat, mw_ref[...]) + mb_ref[0])


def _head(pooled_g, pooled_q, params):
    gp, qp, gru, lp, mp = (params["g_proj"], params["q_proj"], params["gru"],
                           params["lstm_proj"], params["matcher"])
    args = (
        pooled_g[None], pooled_q,
        gp["W"].T, gp["b"][None], qp["W"].T, qp["b"][None],
        gru["W_ih"][0].T, gru["W_hh"][0].T, gru["b_ih"][0][None],
        gru["b_hh"][0][None],
        gru["W_ih"][1].T, gru["W_hh"][1].T, gru["b_ih"][1][None],
        gru["b_hh"][1][None],
        lp["W"].T, lp["b"][None], mp["W"].T, mp["b"][None],
    )
    return pl.pallas_call(
        _head_kernel,
        out_shape=jax.ShapeDtypeStruct((B, 1), jnp.float32),
    )(*args)


def _branch(x0, src, dst, ea_pad, bt3, layers, nq, epad):
    dims = [128, 256, 256]
    xz = jnp.stack([x0[:, :, :64], x0[:, :, 64:]], axis=1)
    scale = jnp.ones((nq, 256), jnp.float32)
    shift = jnp.zeros((nq, 256), jnp.float32)
    pooled = []
    for li, p in enumerate(layers):
        d = dims[li]
        dh = d // 2
        esplit = (li == 0)
        epsf = 1.0 + p["eps"]
        if esplit:
            sc_sc = scale[:, :d].reshape(-1)
            sh_sc = (shift[:, :d] + p["be"][None]).reshape(-1)
            x_sc = x0.reshape(nq * NPAD, d)
            e2 = _embed(ea_pad, p["We"].T, nq, epad, d, True)
            e_sc = e2.reshape(nq * epad, d)
        else:
            sc_sc = scale[:, :d].reshape(-1)
            sh_sc = (shift[:, :d]
                     + p["be"][None]).reshape(-1)
            x_sc = xz.reshape(nq * 2 * NPAD, dh)
            e2 = _embed(ea_pad, p["We"].T, nq, epad, dh, False)
            e_sc = e2.reshape(nq * 2 * epad, dh)
        aggr = _edge_sc(x_sc, e_sc, src, dst, sc_sc, sh_sc,
                        nq, epad, d if esplit else dh, esplit)
        sb = (scale[:, :d] * epsf)[:, None, :]
        hb = (shift[:, :d] * epsf)[:, None, :]
        z, ssum, cnt = _passb(
            xz, aggr.reshape(nq, 2, NPAD, 128), sb, hb,
            p["W1"].T, p["b1"][None], p["W2"].T, p["b2"][None], bt3, nq, d,
            esplit)
        mu = ssum[:, 0, :] / NREAL
        ssq, sseg = _passc(z, mu[:, None, :], bt3, nq)
        var = ssq[:, 0, :] / NREAL
        gamma, beta = p["gamma"][None], p["beta"][None]
        scale = gamma / jnp.sqrt(var + 1e-5)
        shift = beta - mu * scale
        pooled.append(sseg * scale[:, None, :]
                      + cnt[:, 0, :, None] * beta[:, None, :])
        xz = z
    return jnp.concatenate(pooled, axis=-1)


def kernel(crg_x, crg_edge_index, crg_edge_x, crg_batch,
           queries_x, queries_edge_index, queries_edge_x, queries_batch,
           params):
    E = crg_edge_index.shape[1]
    EQ = queries_edge_index.shape[2]
    NQ = queries_x.shape[0]
    epad_g = ((E + 4095) // 4096) * 4096
    epad_q = ((EQ + 4095) // 4096) * 4096

    def prep(x, ei, ea, batch, nq, e, epad):
        xp = jnp.zeros((nq, NPAD, 128), jnp.float32)
        xp = xp.at[:, :NREAL].set(x.reshape(nq, NREAL, 128))
        srcs, dsts, eas = [], [], []
        for i in range(nq):
            s_p, d_p = _pad_edges(ei.reshape(nq, 2, e)[i, 0],
                                  ei.reshape(nq, 2, e)[i, 1], epad)
            srcs.append(s_p)
            dsts.append(d_p)
            eas.append(jnp.pad(ea.reshape(nq, e, 20)[i],
                               ((0, epad - e), (0, 0))))
        bt = jnp.pad(batch.reshape(nq, NREAL), ((0, 0), (0, NPAD - NREAL)))
        bt3 = bt.reshape(nq, NPAD // CH, 1, CH)
        return (xp, jnp.concatenate(srcs), jnp.concatenate(dsts),
                jnp.concatenate(eas), bt3)

    xg, src_g, dst_g, ea_g, bt3_g = prep(
        crg_x, crg_edge_index, crg_edge_x, crg_batch, 1, E, epad_g)
    xq, src_q, dst_q, ea_q, bt3_q = prep(
        queries_x, queries_edge_index, queries_edge_x, queries_batch,
        NQ, EQ, epad_q)

    pooled_g = _branch(xg, src_g, dst_g, ea_g, bt3_g,
                       params["g_layers"], 1, epad_g)[0]
    pooled_q = _branch(xq, src_q, dst_q, ea_q, bt3_q,
                       params["q_layers"], NQ, epad_q)

    return _head(pooled_g, pooled_q, params)
